# Initial kernel scaffold; baseline (speedup 1.0000x reference)
import jax
import jax.numpy as jnp
from jax.experimental import pallas as pl


def kernel(inputs, emb, Wf, Uf, bf, Wb, Ub, bb, Wv, Wd, bd):
    raise NotImplementedError("write your pallas kernel here")


# speedup vs baseline: 3.6181x; 3.6181x over previous
import functools

import jax
import jax.numpy as jnp
from jax import lax
from jax.experimental import pallas as pl
from jax.experimental.pallas import tpu as pltpu
from jax.experimental.pallas import tpu_sc as plsc

B = 1024
T = 200
EMB_D = 64
EMB_P = 128
U = 128

_NC = 2
_NS = 16
_NW = _NC * _NS


def _sc_gather(table, idx):
    n = idx.shape[0]
    d = table.shape[1]
    per_w = n // _NW
    chunk = 400
    n_chunks = per_w // chunk
    mesh = plsc.VectorSubcoreMesh(core_axis_name="c", subcore_axis_name="s")

    @functools.partial(
        pl.kernel,
        mesh=mesh,
        out_type=jax.ShapeDtypeStruct((n, d), jnp.float32),
        scratch_types=[
            pltpu.VMEM((per_w,), jnp.int32),
            pltpu.VMEM((chunk, d), jnp.float32),
            pltpu.VMEM((chunk, d), jnp.float32),
            pltpu.SemaphoreType.DMA,
            pltpu.SemaphoreType.DMA,
        ],
    )
    def k(table_hbm, idx_hbm, out_hbm, idx_v, rows_a, rows_b, sem_a, sem_b):
        wid = lax.axis_index("s") * _NC + lax.axis_index("c")
        base = wid * per_w
        pltpu.sync_copy(idx_hbm.at[pl.ds(base, per_w)], idx_v)
        bufs = (rows_a, rows_b)
        sems = (sem_a, sem_b)
        cps = [None, None]
        for c in range(n_chunks):
            s = c % 2
            cps[s] = pltpu.async_copy(
                table_hbm.at[idx_v.at[pl.ds(c * chunk, chunk)]], bufs[s], sems[s])
            if c > 0:
                cps[1 - s].wait()
                pltpu.sync_copy(bufs[1 - s],
                                out_hbm.at[pl.ds(base + (c - 1) * chunk, chunk)])
        cps[(n_chunks - 1) % 2].wait()
        pltpu.sync_copy(bufs[(n_chunks - 1) % 2],
                        out_hbm.at[pl.ds(base + (n_chunks - 1) * chunk, chunk)])

    return k(table, idx)


def _cell(x, h, c, W, Uk, b):
    z = (jnp.dot(x, W, preferred_element_type=jnp.float32)
         + jnp.dot(h, Uk, preferred_element_type=jnp.float32) + b)
    i = jax.nn.sigmoid(z[:, :U])
    f = jax.nn.sigmoid(z[:, U:2 * U])
    g = jnp.tanh(z[:, 2 * U:3 * U])
    o = jax.nn.sigmoid(z[:, 3 * U:])
    c2 = f * c + i * g
    h2 = o * jnp.tanh(c2)
    return h2, c2


def _scan_body(xf_ref, xb_ref, Wf_ref, Uf_ref, bf_ref, Wb_ref, Ub_ref, bb_ref,
               hf_out, hb_out, ht_out, hf_s, cf_s, hb_s, cb_s):
    t = pl.program_id(0)

    @pl.when(t == 0)
    def _init():
        hf_s[...] = jnp.zeros_like(hf_s)
        cf_s[...] = jnp.zeros_like(cf_s)
        hb_s[...] = jnp.zeros_like(hb_s)
        cb_s[...] = jnp.zeros_like(cb_s)

    h2f, c2f = _cell(xf_ref[0], hf_s[...], cf_s[...], Wf_ref[...], Uf_ref[...],
                     bf_ref[...])
    h2b, c2b = _cell(xb_ref[0], hb_s[...], cb_s[...], Wb_ref[...], Ub_ref[...],
                     bb_ref[...])
    hf_s[...] = h2f
    cf_s[...] = c2f
    hb_s[...] = h2b
    cb_s[...] = c2b
    hf_out[0] = h2f
    hb_out[0] = h2b

    @pl.when(t == 0)
    def _ht0():
        ht_out[:, :U] = h2f
        ht_out[:, U:] = h2b

    @pl.when(t > 0)
    def _htn():
        ht_out[:, :U] = jnp.maximum(ht_out[:, :U], h2f)
        ht_out[:, U:] = jnp.maximum(ht_out[:, U:], h2b)


def _bilstm(x, Wf, Uf, bf, Wb, Ub, bb):
    grid = (T,)
    return pl.pallas_call(
        _scan_body,
        grid=grid,
        in_specs=[
            pl.BlockSpec((1, B, EMB_P), lambda t: (t, 0, 0)),
            pl.BlockSpec((1, B, EMB_P), lambda t: (T - 1 - t, 0, 0)),
            pl.BlockSpec((EMB_P, 4 * U), lambda t: (0, 0)),
            pl.BlockSpec((U, 4 * U), lambda t: (0, 0)),
            pl.BlockSpec((1, 4 * U), lambda t: (0, 0)),
            pl.BlockSpec((EMB_P, 4 * U), lambda t: (0, 0)),
            pl.BlockSpec((U, 4 * U), lambda t: (0, 0)),
            pl.BlockSpec((1, 4 * U), lambda t: (0, 0)),
        ],
        out_specs=[
            pl.BlockSpec((1, B, U), lambda t: (t, 0, 0)),
            pl.BlockSpec((1, B, U), lambda t: (T - 1 - t, 0, 0)),
            pl.BlockSpec((B, 2 * U), lambda t: (0, 0)),
        ],
        out_shape=[
            jax.ShapeDtypeStruct((T, B, U), jnp.float32),
            jax.ShapeDtypeStruct((T, B, U), jnp.float32),
            jax.ShapeDtypeStruct((B, 2 * U), jnp.float32),
        ],
        scratch_shapes=[
            pltpu.VMEM((B, U), jnp.float32),
            pltpu.VMEM((B, U), jnp.float32),
            pltpu.VMEM((B, U), jnp.float32),
            pltpu.VMEM((B, U), jnp.float32),
        ],
        compiler_params=pltpu.CompilerParams(
            dimension_semantics=("arbitrary",)),
    )(x, x, Wf, Uf, bf.reshape(1, -1), Wb, Ub, bb.reshape(1, -1))


def _attn_body(hf_ref, hb_ref, ht_ref, Wv_ref, Wd_ref, bd_ref, out_ref,
               m_s, S_s, Cf_s, Cb_s):
    t = pl.program_id(0)
    hf = hf_ref[0]
    hb = hb_ref[0]
    htf = ht_ref[:, :U]
    htb = ht_ref[:, U:]
    s = (jnp.sum(hf * htf, axis=1, keepdims=True)
         + jnp.sum(hb * htb, axis=1, keepdims=True))

    @pl.when(t == 0)
    def _init():
        m_s[...] = s
        S_s[...] = jnp.ones_like(S_s)
        Cf_s[...] = hf
        Cb_s[...] = hb

    @pl.when(t > 0)
    def _acc():
        m = m_s[...]
        m2 = jnp.maximum(m, s)
        a = jnp.exp(m - m2)
        e = jnp.exp(s - m2)
        m_s[...] = m2
        S_s[...] = S_s[...] * a + e
        Cf_s[...] = Cf_s[...] * a + e * hf
        Cb_s[...] = Cb_s[...] * a + e * hb

    @pl.when(t == T - 1)
    def _head():
        inv = 1.0 / S_s[...]
        ctxf = Cf_s[...] * inv
        ctxb = Cb_s[...] * inv
        z1 = jnp.tanh(
            jnp.dot(ctxf, Wv_ref[:U, :], preferred_element_type=jnp.float32)
            + jnp.dot(ctxb, Wv_ref[U:2 * U, :], preferred_element_type=jnp.float32)
            + jnp.dot(htf, Wv_ref[2 * U:3 * U, :], preferred_element_type=jnp.float32)
            + jnp.dot(htb, Wv_ref[3 * U:, :], preferred_element_type=jnp.float32))
        logits = (jnp.dot(z1, Wd_ref[...], preferred_element_type=jnp.float32)
                  + bd_ref[...])
        mx = jnp.max(logits, axis=1, keepdims=True)
        ex = jnp.exp(logits - mx)
        out_ref[...] = ex / jnp.sum(ex, axis=1, keepdims=True)


def _attention(hf, hb, ht, Wv, Wd, bd):
    grid = (T,)
    return pl.pallas_call(
        _attn_body,
        grid=grid,
        in_specs=[
            pl.BlockSpec((1, B, U), lambda t: (t, 0, 0)),
            pl.BlockSpec((1, B, U), lambda t: (t, 0, 0)),
            pl.BlockSpec((B, 2 * U), lambda t: (0, 0)),
            pl.BlockSpec((4 * U, U), lambda t: (0, 0)),
            pl.BlockSpec((U, 2), lambda t: (0, 0)),
            pl.BlockSpec((1, 2), lambda t: (0, 0)),
        ],
        out_specs=pl.BlockSpec((B, 2), lambda t: (0, 0)),
        out_shape=jax.ShapeDtypeStruct((B, 2), jnp.float32),
        scratch_shapes=[
            pltpu.VMEM((B, 1), jnp.float32),
            pltpu.VMEM((B, 1), jnp.float32),
            pltpu.VMEM((B, U), jnp.float32),
            pltpu.VMEM((B, U), jnp.float32),
        ],
        compiler_params=pltpu.CompilerParams(
            dimension_semantics=("arbitrary",)),
    )(hf, hb, ht, Wv, Wd, bd.reshape(1, -1))


def kernel(inputs, emb, Wf, Uf, bf, Wb, Ub, bb, Wv, Wd, bd):
    idx = jnp.swapaxes(inputs.astype(jnp.int32), 0, 1).reshape(-1)
    pad = EMB_P - EMB_D
    emb_p = jnp.pad(emb, ((0, 0), (0, pad)))
    Wf_p = jnp.pad(Wf, ((0, pad), (0, 0)))
    Wb_p = jnp.pad(Wb, ((0, pad), (0, 0)))
    x = _sc_gather(emb_p, idx).reshape(T, B, EMB_P)
    hf, hb, ht = _bilstm(x, Wf_p, Uf, bf, Wb_p, Ub, bb)
    out = _attention(hf, hb, ht, Wv, Wd, bd)
    return out


# speedup vs baseline: 4.4549x; 1.2313x over previous
import functools

import jax
import jax.numpy as jnp
from jax import lax
from jax.experimental import pallas as pl
from jax.experimental.pallas import tpu as pltpu
from jax.experimental.pallas import tpu_sc as plsc

B = 1024
T = 200
EMB_D = 64
EMB_P = 128
U = 128

_NC = 2
_NS = 16
_NW = _NC * _NS


def _sc_gather(table, idx):
    n = idx.shape[0]
    d = table.shape[1]
    per_w = n // _NW
    chunk = 400
    n_chunks = per_w // chunk
    mesh = plsc.VectorSubcoreMesh(core_axis_name="c", subcore_axis_name="s")

    @functools.partial(
        pl.kernel,
        mesh=mesh,
        out_type=jax.ShapeDtypeStruct((n, d), jnp.float32),
        scratch_types=[
            pltpu.VMEM((per_w,), jnp.int32),
            pltpu.VMEM((chunk, d), jnp.float32),
            pltpu.VMEM((chunk, d), jnp.float32),
            pltpu.SemaphoreType.DMA,
            pltpu.SemaphoreType.DMA,
        ],
    )
    def k(table_hbm, idx_hbm, out_hbm, idx_v, rows_a, rows_b, sem_a, sem_b):
        wid = lax.axis_index("s") * _NC + lax.axis_index("c")
        base = wid * per_w
        pltpu.sync_copy(idx_hbm.at[pl.ds(base, per_w)], idx_v)
        bufs = (rows_a, rows_b)
        sems = (sem_a, sem_b)
        cps = [None, None]
        for c in range(n_chunks):
            s = c % 2
            cps[s] = pltpu.async_copy(
                table_hbm.at[idx_v.at[pl.ds(c * chunk, chunk)]], bufs[s], sems[s])
            if c > 0:
                cps[1 - s].wait()
                pltpu.sync_copy(bufs[1 - s],
                                out_hbm.at[pl.ds(base + (c - 1) * chunk, chunk)])
        cps[(n_chunks - 1) % 2].wait()
        pltpu.sync_copy(bufs[(n_chunks - 1) % 2],
                        out_hbm.at[pl.ds(base + (n_chunks - 1) * chunk, chunk)])

    return k(table, idx)


def _sig(x):
    return 0.5 * jnp.tanh(0.5 * x) + 0.5


def _cell(x_bf, h, c, Wcat_ref, b_ref):
    xcat = jnp.concatenate([x_bf, h.astype(jnp.bfloat16)], axis=1)
    z = jnp.dot(xcat, Wcat_ref[...],
                preferred_element_type=jnp.float32) + b_ref[...]
    i = _sig(z[:, :U])
    f = _sig(z[:, U:2 * U])
    g = jnp.tanh(z[:, 2 * U:3 * U])
    o = _sig(z[:, 3 * U:])
    c2 = f * c + i * g
    h2 = o * jnp.tanh(c2)
    return h2, c2


def _scan_body(xf_ref, xb_ref, Wf_ref, bf_ref, Wb_ref, bb_ref,
               hf_out, hb_out, ht_out, hf_s, cf_s, hb_s, cb_s):
    t = pl.program_id(0)

    @pl.when(t == 0)
    def _init():
        hf_s[...] = jnp.zeros_like(hf_s)
        cf_s[...] = jnp.zeros_like(cf_s)
        hb_s[...] = jnp.zeros_like(hb_s)
        cb_s[...] = jnp.zeros_like(cb_s)

    h2f, c2f = _cell(xf_ref[0].astype(jnp.bfloat16), hf_s[...], cf_s[...],
                     Wf_ref, bf_ref)
    h2b, c2b = _cell(xb_ref[0].astype(jnp.bfloat16), hb_s[...], cb_s[...],
                     Wb_ref, bb_ref)
    hf_s[...] = h2f
    cf_s[...] = c2f
    hb_s[...] = h2b
    cb_s[...] = c2b
    hf_out[0] = h2f.astype(jnp.bfloat16)
    hb_out[0] = h2b.astype(jnp.bfloat16)

    @pl.when(t == 0)
    def _ht0():
        ht_out[:, :U] = h2f
        ht_out[:, U:] = h2b

    @pl.when(t > 0)
    def _htn():
        ht_out[:, :U] = jnp.maximum(ht_out[:, :U], h2f)
        ht_out[:, U:] = jnp.maximum(ht_out[:, U:], h2b)


def _bilstm(x, Wcf, bf, Wcb, bb):
    grid = (T,)
    return pl.pallas_call(
        _scan_body,
        grid=grid,
        in_specs=[
            pl.BlockSpec((1, B, EMB_P), lambda t: (t, 0, 0)),
            pl.BlockSpec((1, B, EMB_P), lambda t: (T - 1 - t, 0, 0)),
            pl.BlockSpec((EMB_P + U, 4 * U), lambda t: (0, 0)),
            pl.BlockSpec((1, 4 * U), lambda t: (0, 0)),
            pl.BlockSpec((EMB_P + U, 4 * U), lambda t: (0, 0)),
            pl.BlockSpec((1, 4 * U), lambda t: (0, 0)),
        ],
        out_specs=[
            pl.BlockSpec((1, B, U), lambda t: (t, 0, 0)),
            pl.BlockSpec((1, B, U), lambda t: (T - 1 - t, 0, 0)),
            pl.BlockSpec((B, 2 * U), lambda t: (0, 0)),
        ],
        out_shape=[
            jax.ShapeDtypeStruct((T, B, U), jnp.bfloat16),
            jax.ShapeDtypeStruct((T, B, U), jnp.bfloat16),
            jax.ShapeDtypeStruct((B, 2 * U), jnp.float32),
        ],
        scratch_shapes=[
            pltpu.VMEM((B, U), jnp.float32),
            pltpu.VMEM((B, U), jnp.float32),
            pltpu.VMEM((B, U), jnp.float32),
            pltpu.VMEM((B, U), jnp.float32),
        ],
        compiler_params=pltpu.CompilerParams(
            dimension_semantics=("arbitrary",)),
    )(x, x, Wcf, bf.reshape(1, -1), Wcb, bb.reshape(1, -1))


def _attn_body(hf_ref, hb_ref, ht_ref, Wv_ref, Wd_ref, bd_ref, out_ref,
               m_s, S_s, Cf_s, Cb_s):
    t = pl.program_id(0)
    hf = hf_ref[0].astype(jnp.float32)
    hb = hb_ref[0].astype(jnp.float32)
    htf = ht_ref[:, :U]
    htb = ht_ref[:, U:]
    s = (jnp.sum(hf * htf, axis=1, keepdims=True)
         + jnp.sum(hb * htb, axis=1, keepdims=True))

    @pl.when(t == 0)
    def _init():
        m_s[...] = s
        S_s[...] = jnp.ones_like(S_s)
        Cf_s[...] = hf
        Cb_s[...] = hb

    @pl.when(t > 0)
    def _acc():
        m = m_s[...]
        m2 = jnp.maximum(m, s)
        a = jnp.exp(m - m2)
        e = jnp.exp(s - m2)
        m_s[...] = m2
        S_s[...] = S_s[...] * a + e
        Cf_s[...] = Cf_s[...] * a + e * hf
        Cb_s[...] = Cb_s[...] * a + e * hb

    @pl.when(t == T - 1)
    def _head():
        inv = 1.0 / S_s[...]
        ctxf = Cf_s[...] * inv
        ctxb = Cb_s[...] * inv
        z1 = jnp.tanh(
            jnp.dot(ctxf, Wv_ref[:U, :], preferred_element_type=jnp.float32)
            + jnp.dot(ctxb, Wv_ref[U:2 * U, :], preferred_element_type=jnp.float32)
            + jnp.dot(htf, Wv_ref[2 * U:3 * U, :], preferred_element_type=jnp.float32)
            + jnp.dot(htb, Wv_ref[3 * U:, :], preferred_element_type=jnp.float32))
        logits = (jnp.dot(z1, Wd_ref[...], preferred_element_type=jnp.float32)
                  + bd_ref[...])
        mx = jnp.max(logits, axis=1, keepdims=True)
        ex = jnp.exp(logits - mx)
        out_ref[...] = ex / jnp.sum(ex, axis=1, keepdims=True)


def _attention(hf, hb, ht, Wv, Wd, bd):
    grid = (T,)
    return pl.pallas_call(
        _attn_body,
        grid=grid,
        in_specs=[
            pl.BlockSpec((1, B, U), lambda t: (t, 0, 0)),
            pl.BlockSpec((1, B, U), lambda t: (t, 0, 0)),
            pl.BlockSpec((B, 2 * U), lambda t: (0, 0)),
            pl.BlockSpec((4 * U, U), lambda t: (0, 0)),
            pl.BlockSpec((U, 2), lambda t: (0, 0)),
            pl.BlockSpec((1, 2), lambda t: (0, 0)),
        ],
        out_specs=pl.BlockSpec((B, 2), lambda t: (0, 0)),
        out_shape=jax.ShapeDtypeStruct((B, 2), jnp.float32),
        scratch_shapes=[
            pltpu.VMEM((B, 1), jnp.float32),
            pltpu.VMEM((B, 1), jnp.float32),
            pltpu.VMEM((B, U), jnp.float32),
            pltpu.VMEM((B, U), jnp.float32),
        ],
        compiler_params=pltpu.CompilerParams(
            dimension_semantics=("arbitrary",)),
    )(hf, hb, ht, Wv, Wd, bd.reshape(1, -1))


def kernel(inputs, emb, Wf, Uf, bf, Wb, Ub, bb, Wv, Wd, bd):
    idx = jnp.swapaxes(inputs.astype(jnp.int32), 0, 1).reshape(-1)
    pad = EMB_P - EMB_D
    emb_p = jnp.pad(emb, ((0, 0), (0, pad)))
    Wcf = jnp.concatenate(
        [jnp.pad(Wf, ((0, pad), (0, 0))), Uf], axis=0).astype(jnp.bfloat16)
    Wcb = jnp.concatenate(
        [jnp.pad(Wb, ((0, pad), (0, 0))), Ub], axis=0).astype(jnp.bfloat16)
    x = _sc_gather(emb_p, idx).reshape(T, B, EMB_P)
    hf, hb, ht = _bilstm(x, Wcf, bf, Wcb, bb)
    out = _attention(hf, hb, ht, Wv, Wd, bd)
    return out


# speedup vs baseline: 4.5167x; 1.0139x over previous
import functools

import jax
import jax.numpy as jnp
from jax import lax
from jax.experimental import pallas as pl
from jax.experimental.pallas import tpu as pltpu
from jax.experimental.pallas import tpu_sc as plsc

B = 1024
T = 200
EMB_D = 64
EMB_P = 128
U = 128

_NC = 2
_NS = 16
_NW = _NC * _NS


def _sc_gather(table, idx):
    n = idx.shape[0]
    d = table.shape[1]
    per_w = n // _NW
    chunk = 400
    n_chunks = per_w // chunk
    mesh = plsc.VectorSubcoreMesh(core_axis_name="c", subcore_axis_name="s")

    @functools.partial(
        pl.kernel,
        mesh=mesh,
        out_type=jax.ShapeDtypeStruct((n, d), jnp.float32),
        scratch_types=[
            pltpu.VMEM((per_w,), jnp.int32),
            pltpu.VMEM((chunk, d), jnp.float32),
            pltpu.VMEM((chunk, d), jnp.float32),
            pltpu.SemaphoreType.DMA,
            pltpu.SemaphoreType.DMA,
        ],
    )
    def k(table_hbm, idx_hbm, out_hbm, idx_v, rows_a, rows_b, sem_a, sem_b):
        wid = lax.axis_index("s") * _NC + lax.axis_index("c")
        base = wid * per_w
        pltpu.sync_copy(idx_hbm.at[pl.ds(base, per_w)], idx_v)
        bufs = (rows_a, rows_b)
        sems = (sem_a, sem_b)
        cps = [None, None]
        for c in range(n_chunks):
            s = c % 2
            cps[s] = pltpu.async_copy(
                table_hbm.at[idx_v.at[pl.ds(c * chunk, chunk)]], bufs[s], sems[s])
            if c > 0:
                cps[1 - s].wait()
                pltpu.sync_copy(bufs[1 - s],
                                out_hbm.at[pl.ds(base + (c - 1) * chunk, chunk)])
        cps[(n_chunks - 1) % 2].wait()
        pltpu.sync_copy(bufs[(n_chunks - 1) % 2],
                        out_hbm.at[pl.ds(base + (n_chunks - 1) * chunk, chunk)])

    return k(table, idx)


def _cell(x_bf, h, c, Wcat_ref, b_ref):
    xcat = jnp.concatenate([x_bf, h.astype(jnp.bfloat16)], axis=1)
    z = jnp.dot(xcat, Wcat_ref[...],
                preferred_element_type=jnp.float32) + b_ref[...]
    i = 0.5 * jnp.tanh(z[:, :U]) + 0.5
    f = 0.5 * jnp.tanh(z[:, U:2 * U]) + 0.5
    g = jnp.tanh(z[:, 2 * U:3 * U])
    o = 0.5 * jnp.tanh(z[:, 3 * U:]) + 0.5
    c2 = f * c + i * g
    h2 = o * jnp.tanh(c2)
    return h2, c2


def _scan_body(xf_ref, xb_ref, Wf_ref, bf_ref, Wb_ref, bb_ref,
               hf_out, hb_out, ht_out, hf_s, cf_s, hb_s, cb_s):
    t = pl.program_id(0)

    @pl.when(t == 0)
    def _init():
        hf_s[...] = jnp.zeros_like(hf_s)
        cf_s[...] = jnp.zeros_like(cf_s)
        hb_s[...] = jnp.zeros_like(hb_s)
        cb_s[...] = jnp.zeros_like(cb_s)

    h2f, c2f = _cell(xf_ref[0].astype(jnp.bfloat16), hf_s[...], cf_s[...],
                     Wf_ref, bf_ref)
    h2b, c2b = _cell(xb_ref[0].astype(jnp.bfloat16), hb_s[...], cb_s[...],
                     Wb_ref, bb_ref)
    hf_s[...] = h2f
    cf_s[...] = c2f
    hb_s[...] = h2b
    cb_s[...] = c2b
    hf_out[0] = h2f.astype(jnp.bfloat16)
    hb_out[0] = h2b.astype(jnp.bfloat16)

    @pl.when(t == 0)
    def _ht0():
        ht_out[:, :U] = h2f
        ht_out[:, U:] = h2b

    @pl.when(t > 0)
    def _htn():
        ht_out[:, :U] = jnp.maximum(ht_out[:, :U], h2f)
        ht_out[:, U:] = jnp.maximum(ht_out[:, U:], h2b)


def _bilstm(x, Wcf, bf, Wcb, bb):
    grid = (T,)
    return pl.pallas_call(
        _scan_body,
        grid=grid,
        in_specs=[
            pl.BlockSpec((1, B, EMB_P), lambda t: (t, 0, 0)),
            pl.BlockSpec((1, B, EMB_P), lambda t: (T - 1 - t, 0, 0)),
            pl.BlockSpec((EMB_P + U, 4 * U), lambda t: (0, 0)),
            pl.BlockSpec((1, 4 * U), lambda t: (0, 0)),
            pl.BlockSpec((EMB_P + U, 4 * U), lambda t: (0, 0)),
            pl.BlockSpec((1, 4 * U), lambda t: (0, 0)),
        ],
        out_specs=[
            pl.BlockSpec((1, B, U), lambda t: (t, 0, 0)),
            pl.BlockSpec((1, B, U), lambda t: (T - 1 - t, 0, 0)),
            pl.BlockSpec((B, 2 * U), lambda t: (0, 0)),
        ],
        out_shape=[
            jax.ShapeDtypeStruct((T, B, U), jnp.bfloat16),
            jax.ShapeDtypeStruct((T, B, U), jnp.bfloat16),
            jax.ShapeDtypeStruct((B, 2 * U), jnp.float32),
        ],
        scratch_shapes=[
            pltpu.VMEM((B, U), jnp.float32),
            pltpu.VMEM((B, U), jnp.float32),
            pltpu.VMEM((B, U), jnp.float32),
            pltpu.VMEM((B, U), jnp.float32),
        ],
        compiler_params=pltpu.CompilerParams(
            dimension_semantics=("arbitrary",)),
    )(x, x, Wcf, bf.reshape(1, -1), Wcb, bb.reshape(1, -1))


def _attn_body(hf_ref, hb_ref, ht_ref, ones_ref, Wv_ref, Wd_ref, bd_ref,
               out_ref, m_s, S_s, Cf_s, Cb_s, htbf_s):
    t = pl.program_id(0)
    hf = hf_ref[0]
    hb = hb_ref[0]

    @pl.when(t == 0)
    def _cast_ht():
        htbf_s[...] = ht_ref[...].astype(jnp.bfloat16)

    qf = hf * htbf_s[:, :U]
    qb = hb * htbf_s[:, U:]
    s = (jnp.dot(qf, ones_ref[...], preferred_element_type=jnp.float32)
         + jnp.dot(qb, ones_ref[...], preferred_element_type=jnp.float32))

    @pl.when(t == 0)
    def _init():
        m_s[...] = s
        S_s[...] = jnp.ones_like(S_s)
        Cf_s[...] = hf.astype(jnp.float32)
        Cb_s[...] = hb.astype(jnp.float32)

    @pl.when(t > 0)
    def _acc():
        m = m_s[...]
        m2 = jnp.maximum(m, s)
        a = jnp.exp(m - m2)
        e = jnp.exp(s - m2)
        m_s[...] = m2
        S_s[...] = S_s[...] * a + e
        Cf_s[...] = Cf_s[...] * a + e * hf.astype(jnp.float32)
        Cb_s[...] = Cb_s[...] * a + e * hb.astype(jnp.float32)

    @pl.when(t == T - 1)
    def _head():
        htf = ht_ref[:, :U]
        htb = ht_ref[:, U:]
        inv = 1.0 / S_s[...]
        ctxf = Cf_s[...] * inv
        ctxb = Cb_s[...] * inv
        z1 = jnp.tanh(
            jnp.dot(ctxf, Wv_ref[:U, :], preferred_element_type=jnp.float32)
            + jnp.dot(ctxb, Wv_ref[U:2 * U, :], preferred_element_type=jnp.float32)
            + jnp.dot(htf, Wv_ref[2 * U:3 * U, :], preferred_element_type=jnp.float32)
            + jnp.dot(htb, Wv_ref[3 * U:, :], preferred_element_type=jnp.float32))
        logits = (jnp.dot(z1, Wd_ref[...], preferred_element_type=jnp.float32)
                  + bd_ref[...])
        mx = jnp.max(logits, axis=1, keepdims=True)
        ex = jnp.exp(logits - mx)
        out_ref[...] = ex / jnp.sum(ex, axis=1, keepdims=True)


def _attention(hf, hb, ht, Wv, Wd, bd):
    grid = (T,)
    ones = jnp.ones((U, U), jnp.bfloat16)
    return pl.pallas_call(
        _attn_body,
        grid=grid,
        in_specs=[
            pl.BlockSpec((1, B, U), lambda t: (t, 0, 0)),
            pl.BlockSpec((1, B, U), lambda t: (t, 0, 0)),
            pl.BlockSpec((B, 2 * U), lambda t: (0, 0)),
            pl.BlockSpec((U, U), lambda t: (0, 0)),
            pl.BlockSpec((4 * U, U), lambda t: (0, 0)),
            pl.BlockSpec((U, 2), lambda t: (0, 0)),
            pl.BlockSpec((1, 2), lambda t: (0, 0)),
        ],
        out_specs=pl.BlockSpec((B, 2), lambda t: (0, 0)),
        out_shape=jax.ShapeDtypeStruct((B, 2), jnp.float32),
        scratch_shapes=[
            pltpu.VMEM((B, U), jnp.float32),
            pltpu.VMEM((B, U), jnp.float32),
            pltpu.VMEM((B, U), jnp.float32),
            pltpu.VMEM((B, U), jnp.float32),
            pltpu.VMEM((B, 2 * U), jnp.bfloat16),
        ],
        compiler_params=pltpu.CompilerParams(
            dimension_semantics=("arbitrary",)),
    )(hf, hb, ht, ones, Wv, Wd, bd.reshape(1, -1))


def kernel(inputs, emb, Wf, Uf, bf, Wb, Ub, bb, Wv, Wd, bd):
    idx = jnp.swapaxes(inputs.astype(jnp.int32), 0, 1).reshape(-1)
    pad = EMB_P - EMB_D
    emb_p = jnp.pad(emb, ((0, 0), (0, pad)))
    gate_scale = jnp.concatenate(
        [jnp.full((2 * U,), 0.5), jnp.ones((U,)), jnp.full((U,), 0.5)]
    ).astype(jnp.float32)
    Wcf = (jnp.concatenate([jnp.pad(Wf, ((0, pad), (0, 0))), Uf], axis=0)
           * gate_scale).astype(jnp.bfloat16)
    Wcb = (jnp.concatenate([jnp.pad(Wb, ((0, pad), (0, 0))), Ub], axis=0)
           * gate_scale).astype(jnp.bfloat16)
    x = _sc_gather(emb_p, idx).reshape(T, B, EMB_P)
    hf, hb, ht = _bilstm(x, Wcf, bf * gate_scale, Wcb, bb * gate_scale)
    out = _attention(hf, hb, ht, Wv, Wd, bd)
    return out


# speedup vs baseline: 4.6925x; 1.0389x over previous
import functools

import jax
import jax.numpy as jnp
from jax import lax
from jax.experimental import pallas as pl
from jax.experimental.pallas import tpu as pltpu
from jax.experimental.pallas import tpu_sc as plsc

B = 1024
T = 200
EMB_D = 64
EMB_P = 128
U = 128

_NC = 2
_NS = 16
_NW = _NC * _NS


def _sc_gather(table, idx):
    n = idx.shape[0]
    d = table.shape[1]
    per_w = n // _NW
    chunk = 400
    n_chunks = per_w // chunk
    mesh = plsc.VectorSubcoreMesh(core_axis_name="c", subcore_axis_name="s")

    @functools.partial(
        pl.kernel,
        mesh=mesh,
        out_type=jax.ShapeDtypeStruct((n, d), jnp.float32),
        scratch_types=[
            pltpu.VMEM((per_w,), jnp.int32),
            pltpu.VMEM((chunk, d), jnp.float32),
            pltpu.VMEM((chunk, d), jnp.float32),
            pltpu.SemaphoreType.DMA,
            pltpu.SemaphoreType.DMA,
        ],
    )
    def k(table_hbm, idx_hbm, out_hbm, idx_v, rows_a, rows_b, sem_a, sem_b):
        wid = lax.axis_index("s") * _NC + lax.axis_index("c")
        base = wid * per_w
        pltpu.sync_copy(idx_hbm.at[pl.ds(base, per_w)], idx_v)
        bufs = (rows_a, rows_b)
        sems = (sem_a, sem_b)
        cps = [None, None]
        for c in range(n_chunks):
            s = c % 2
            cps[s] = pltpu.async_copy(
                table_hbm.at[idx_v.at[pl.ds(c * chunk, chunk)]], bufs[s], sems[s])
            if c > 0:
                cps[1 - s].wait()
                pltpu.sync_copy(bufs[1 - s],
                                out_hbm.at[pl.ds(base + (c - 1) * chunk, chunk)])
        cps[(n_chunks - 1) % 2].wait()
        pltpu.sync_copy(bufs[(n_chunks - 1) % 2],
                        out_hbm.at[pl.ds(base + (n_chunks - 1) * chunk, chunk)])

    return k(table, idx)


def _cell(x_bf, h_bf, c_bf, Wcat_ref, b_ref):
    xcat = jnp.concatenate([x_bf, h_bf], axis=1)
    z = (jnp.dot(xcat, Wcat_ref[...], preferred_element_type=jnp.float32)
         + b_ref[...]).astype(jnp.bfloat16)
    half = jnp.bfloat16(0.5)
    i = half * jnp.tanh(z[:, :U]) + half
    f = half * jnp.tanh(z[:, U:2 * U]) + half
    g = jnp.tanh(z[:, 2 * U:3 * U])
    o = half * jnp.tanh(z[:, 3 * U:]) + half
    c2 = f * c_bf + i * g
    h2 = o * jnp.tanh(c2)
    return h2, c2


def _scan_body(xf_ref, xb_ref, Wf_ref, bf_ref, Wb_ref, bb_ref,
               hf_out, hb_out, ht_out, hf_s, cf_s, hb_s, cb_s):
    t = pl.program_id(0)

    @pl.when(t == 0)
    def _init():
        hf_s[...] = jnp.zeros_like(hf_s)
        cf_s[...] = jnp.zeros_like(cf_s)
        hb_s[...] = jnp.zeros_like(hb_s)
        cb_s[...] = jnp.zeros_like(cb_s)

    h2f, c2f = _cell(xf_ref[0].astype(jnp.bfloat16), hf_s[...], cf_s[...],
                     Wf_ref, bf_ref)
    h2b, c2b = _cell(xb_ref[0].astype(jnp.bfloat16), hb_s[...], cb_s[...],
                     Wb_ref, bb_ref)
    hf_s[...] = h2f
    cf_s[...] = c2f
    hb_s[...] = h2b
    cb_s[...] = c2b
    hf_out[0] = h2f
    hb_out[0] = h2b

    @pl.when(t == 0)
    def _ht0():
        ht_out[:, :U] = h2f.astype(jnp.float32)
        ht_out[:, U:] = h2b.astype(jnp.float32)

    @pl.when(t > 0)
    def _htn():
        ht_out[:, :U] = jnp.maximum(ht_out[:, :U], h2f.astype(jnp.float32))
        ht_out[:, U:] = jnp.maximum(ht_out[:, U:], h2b.astype(jnp.float32))


def _bilstm(x, Wcf, bf, Wcb, bb):
    grid = (T,)
    return pl.pallas_call(
        _scan_body,
        grid=grid,
        in_specs=[
            pl.BlockSpec((1, B, EMB_P), lambda t: (t, 0, 0)),
            pl.BlockSpec((1, B, EMB_P), lambda t: (T - 1 - t, 0, 0)),
            pl.BlockSpec((EMB_P + U, 4 * U), lambda t: (0, 0)),
            pl.BlockSpec((1, 4 * U), lambda t: (0, 0)),
            pl.BlockSpec((EMB_P + U, 4 * U), lambda t: (0, 0)),
            pl.BlockSpec((1, 4 * U), lambda t: (0, 0)),
        ],
        out_specs=[
            pl.BlockSpec((1, B, U), lambda t: (t, 0, 0)),
            pl.BlockSpec((1, B, U), lambda t: (T - 1 - t, 0, 0)),
            pl.BlockSpec((B, 2 * U), lambda t: (0, 0)),
        ],
        out_shape=[
            jax.ShapeDtypeStruct((T, B, U), jnp.bfloat16),
            jax.ShapeDtypeStruct((T, B, U), jnp.bfloat16),
            jax.ShapeDtypeStruct((B, 2 * U), jnp.float32),
        ],
        scratch_shapes=[
            pltpu.VMEM((B, U), jnp.bfloat16),
            pltpu.VMEM((B, U), jnp.bfloat16),
            pltpu.VMEM((B, U), jnp.bfloat16),
            pltpu.VMEM((B, U), jnp.bfloat16),
        ],
        compiler_params=pltpu.CompilerParams(
            dimension_semantics=("arbitrary",)),
    )(x, x, Wcf, bf.reshape(1, -1), Wcb, bb.reshape(1, -1))


def _attn_body(hf_ref, hb_ref, ht_ref, ones_ref, Wv_ref, Wd_ref, bd_ref,
               out_ref, S_s, Cf_s, Cb_s, htbf_s):
    t = pl.program_id(0)
    hf = hf_ref[0]
    hb = hb_ref[0]

    @pl.when(t == 0)
    def _cast_ht():
        htbf_s[...] = ht_ref[...].astype(jnp.bfloat16)

    qf = hf * htbf_s[:, :U]
    qb = hb * htbf_s[:, U:]
    s = (jnp.dot(qf, ones_ref[...], preferred_element_type=jnp.float32)
         + jnp.dot(qb, ones_ref[...], preferred_element_type=jnp.float32))
    e = jnp.exp(jnp.minimum(s, 80.0))

    @pl.when(t == 0)
    def _init():
        S_s[...] = e
        Cf_s[...] = e * hf.astype(jnp.float32)
        Cb_s[...] = e * hb.astype(jnp.float32)

    @pl.when(t > 0)
    def _acc():
        S_s[...] = S_s[...] + e
        Cf_s[...] = Cf_s[...] + e * hf.astype(jnp.float32)
        Cb_s[...] = Cb_s[...] + e * hb.astype(jnp.float32)

    @pl.when(t == T - 1)
    def _head():
        htf = ht_ref[:, :U]
        htb = ht_ref[:, U:]
        inv = 1.0 / S_s[...]
        ctxf = Cf_s[...] * inv
        ctxb = Cb_s[...] * inv
        z1 = jnp.tanh(
            jnp.dot(ctxf, Wv_ref[:U, :], preferred_element_type=jnp.float32)
            + jnp.dot(ctxb, Wv_ref[U:2 * U, :], preferred_element_type=jnp.float32)
            + jnp.dot(htf, Wv_ref[2 * U:3 * U, :], preferred_element_type=jnp.float32)
            + jnp.dot(htb, Wv_ref[3 * U:, :], preferred_element_type=jnp.float32))
        logits = (jnp.dot(z1, Wd_ref[...], preferred_element_type=jnp.float32)
                  + bd_ref[...])
        mx = jnp.max(logits, axis=1, keepdims=True)
        ex = jnp.exp(logits - mx)
        out_ref[...] = ex / jnp.sum(ex, axis=1, keepdims=True)


def _attention(hf, hb, ht, Wv, Wd, bd):
    grid = (T,)
    ones = jnp.ones((U, U), jnp.bfloat16)
    return pl.pallas_call(
        _attn_body,
        grid=grid,
        in_specs=[
            pl.BlockSpec((1, B, U), lambda t: (t, 0, 0)),
            pl.BlockSpec((1, B, U), lambda t: (t, 0, 0)),
            pl.BlockSpec((B, 2 * U), lambda t: (0, 0)),
            pl.BlockSpec((U, U), lambda t: (0, 0)),
            pl.BlockSpec((4 * U, U), lambda t: (0, 0)),
            pl.BlockSpec((U, 2), lambda t: (0, 0)),
            pl.BlockSpec((1, 2), lambda t: (0, 0)),
        ],
        out_specs=pl.BlockSpec((B, 2), lambda t: (0, 0)),
        out_shape=jax.ShapeDtypeStruct((B, 2), jnp.float32),
        scratch_shapes=[
            pltpu.VMEM((B, U), jnp.float32),
            pltpu.VMEM((B, U), jnp.float32),
            pltpu.VMEM((B, U), jnp.float32),
            pltpu.VMEM((B, 2 * U), jnp.bfloat16),
        ],
        compiler_params=pltpu.CompilerParams(
            dimension_semantics=("arbitrary",)),
    )(hf, hb, ht, ones, Wv, Wd, bd.reshape(1, -1))


def kernel(inputs, emb, Wf, Uf, bf, Wb, Ub, bb, Wv, Wd, bd):
    idx = jnp.swapaxes(inputs.astype(jnp.int32), 0, 1).reshape(-1)
    pad = EMB_P - EMB_D
    emb_p = jnp.pad(emb, ((0, 0), (0, pad)))
    gate_scale = jnp.concatenate(
        [jnp.full((2 * U,), 0.5), jnp.ones((U,)), jnp.full((U,), 0.5)]
    ).astype(jnp.float32)
    Wcf = (jnp.concatenate([jnp.pad(Wf, ((0, pad), (0, 0))), Uf], axis=0)
           * gate_scale).astype(jnp.bfloat16)
    Wcb = (jnp.concatenate([jnp.pad(Wb, ((0, pad), (0, 0))), Ub], axis=0)
           * gate_scale).astype(jnp.bfloat16)
    x = _sc_gather(emb_p, idx).reshape(T, B, EMB_P)
    hf, hb, ht = _bilstm(x, Wcf, bf * gate_scale, Wcb, bb * gate_scale)
    out = _attention(hf, hb, ht, Wv, Wd, bd)
    return out


# speedup vs baseline: 5.6187x; 1.1974x over previous
import functools

import jax
import jax.numpy as jnp
from jax import lax
from jax.experimental import pallas as pl
from jax.experimental.pallas import tpu as pltpu
from jax.experimental.pallas import tpu_sc as plsc

B = 1024
T = 200
EMB_D = 64
EMB_P = 128
U = 128

_NC = 2
_NS = 16
_NW = _NC * _NS


def _sc_gather(table, idx):
    n = idx.shape[0]
    d = table.shape[1]
    dt = table.dtype
    per_w = n // _NW
    chunk = 800 if dt == jnp.bfloat16 else 400
    n_chunks = per_w // chunk
    mesh = plsc.VectorSubcoreMesh(core_axis_name="c", subcore_axis_name="s")

    @functools.partial(
        pl.kernel,
        mesh=mesh,
        out_type=jax.ShapeDtypeStruct((n, d), dt),
        scratch_types=[
            pltpu.VMEM((per_w,), jnp.int32),
            pltpu.VMEM((chunk, d), dt),
            pltpu.VMEM((chunk, d), dt),
            pltpu.SemaphoreType.DMA,
            pltpu.SemaphoreType.DMA,
        ],
    )
    def k(table_hbm, idx_hbm, out_hbm, idx_v, rows_a, rows_b, sem_a, sem_b):
        wid = lax.axis_index("s") * _NC + lax.axis_index("c")
        base = wid * per_w
        pltpu.sync_copy(idx_hbm.at[pl.ds(base, per_w)], idx_v)
        bufs = (rows_a, rows_b)
        sems = (sem_a, sem_b)
        cps = [None, None]
        for c in range(n_chunks):
            s = c % 2
            cps[s] = pltpu.async_copy(
                table_hbm.at[idx_v.at[pl.ds(c * chunk, chunk)]], bufs[s], sems[s])
            if c > 0:
                cps[1 - s].wait()
                pltpu.sync_copy(bufs[1 - s],
                                out_hbm.at[pl.ds(base + (c - 1) * chunk, chunk)])
        cps[(n_chunks - 1) % 2].wait()
        pltpu.sync_copy(bufs[(n_chunks - 1) % 2],
                        out_hbm.at[pl.ds(base + (n_chunks - 1) * chunk, chunk)])

    return k(table, idx)


def _cell(x_bf, h_bf, c_bf, Wcat_ref, b_ref):
    xcat = jnp.concatenate([x_bf, h_bf], axis=1)
    z = jnp.dot(xcat, Wcat_ref[...],
                preferred_element_type=jnp.float32).astype(jnp.bfloat16)
    z = z + b_ref[...]
    half = jnp.bfloat16(0.5)
    i = half * jnp.tanh(z[:, :U]) + half
    f = half * jnp.tanh(z[:, U:2 * U]) + half
    g = jnp.tanh(z[:, 2 * U:3 * U])
    o = half * jnp.tanh(z[:, 3 * U:]) + half
    c2 = f * c_bf + i * g
    h2 = o * jnp.tanh(c2)
    return h2, c2


def _scan_body(xf_ref, xb_ref, Wf_ref, bf_ref, Wb_ref, bb_ref,
               hf_out, hb_out, ht_out, hf_s, cf_s, hb_s, cb_s):
    t = pl.program_id(0)

    @pl.when(t == 0)
    def _init():
        hf_s[...] = jnp.zeros_like(hf_s)
        cf_s[...] = jnp.zeros_like(cf_s)
        hb_s[...] = jnp.zeros_like(hb_s)
        cb_s[...] = jnp.zeros_like(cb_s)

    h2f, c2f = _cell(xf_ref[0].astype(jnp.bfloat16), hf_s[...], cf_s[...],
                     Wf_ref, bf_ref)
    h2b, c2b = _cell(xb_ref[0].astype(jnp.bfloat16), hb_s[...], cb_s[...],
                     Wb_ref, bb_ref)
    hf_s[...] = h2f
    cf_s[...] = c2f
    hb_s[...] = h2b
    cb_s[...] = c2b
    hf_out[0] = h2f
    hb_out[0] = h2b

    @pl.when(t == 0)
    def _ht0():
        ht_out[:, :U] = h2f
        ht_out[:, U:] = h2b

    @pl.when(t > 0)
    def _htn():
        ht_out[:, :U] = jnp.maximum(ht_out[:, :U], h2f)
        ht_out[:, U:] = jnp.maximum(ht_out[:, U:], h2b)


def _bilstm(x, Wcf, bf, Wcb, bb):
    grid = (T,)
    return pl.pallas_call(
        _scan_body,
        grid=grid,
        in_specs=[
            pl.BlockSpec((1, B, EMB_P), lambda t: (t, 0, 0)),
            pl.BlockSpec((1, B, EMB_P), lambda t: (T - 1 - t, 0, 0)),
            pl.BlockSpec((EMB_P + U, 4 * U), lambda t: (0, 0)),
            pl.BlockSpec((1, 4 * U), lambda t: (0, 0)),
            pl.BlockSpec((EMB_P + U, 4 * U), lambda t: (0, 0)),
            pl.BlockSpec((1, 4 * U), lambda t: (0, 0)),
        ],
        out_specs=[
            pl.BlockSpec((1, B, U), lambda t: (t, 0, 0)),
            pl.BlockSpec((1, B, U), lambda t: (T - 1 - t, 0, 0)),
            pl.BlockSpec((B, 2 * U), lambda t: (0, 0)),
        ],
        out_shape=[
            jax.ShapeDtypeStruct((T, B, U), jnp.bfloat16),
            jax.ShapeDtypeStruct((T, B, U), jnp.bfloat16),
            jax.ShapeDtypeStruct((B, 2 * U), jnp.bfloat16),
        ],
        scratch_shapes=[
            pltpu.VMEM((B, U), jnp.bfloat16),
            pltpu.VMEM((B, U), jnp.bfloat16),
            pltpu.VMEM((B, U), jnp.bfloat16),
            pltpu.VMEM((B, U), jnp.bfloat16),
        ],
        compiler_params=pltpu.CompilerParams(
            dimension_semantics=("arbitrary",)),
    )(x, x, Wcf, bf.reshape(1, -1).astype(jnp.bfloat16),
      Wcb, bb.reshape(1, -1).astype(jnp.bfloat16))


TW = 4


def _attn_body(hf_ref, hb_ref, ht_ref, ones_ref, Wv_ref, Wd_ref, bd_ref,
               out_ref, S_s, Cf_s, Cb_s, q_s):
    tb = pl.program_id(0)

    @pl.when(tb == 0)
    def _zero():
        S_s[...] = jnp.zeros_like(S_s)
        Cf_s[...] = jnp.zeros_like(Cf_s)
        Cb_s[...] = jnp.zeros_like(Cb_s)

    es, hfs, hbs = [], [], []
    for k in range(TW):
        hfk = hf_ref[k]
        hbk = hb_ref[k]
        q_s[:, :U] = hfk * ht_ref[:, :U]
        q_s[:, U:] = hbk * ht_ref[:, U:]
        s = jnp.dot(q_s[...], ones_ref[...],
                    preferred_element_type=jnp.float32)
        es.append(jnp.exp(jnp.minimum(s, 80.0)))
        hfs.append(hfk.astype(jnp.float32))
        hbs.append(hbk.astype(jnp.float32))

    S_s[...] = S_s[...] + sum(es)
    Cf_s[...] = Cf_s[...] + sum(e * h for e, h in zip(es, hfs))
    Cb_s[...] = Cb_s[...] + sum(e * h for e, h in zip(es, hbs))

    @pl.when(tb == T // TW - 1)
    def _head():
        htf = ht_ref[:, :U].astype(jnp.float32)
        htb = ht_ref[:, U:].astype(jnp.float32)
        inv = 1.0 / S_s[...]
        ctxf = Cf_s[...] * inv
        ctxb = Cb_s[...] * inv
        z1 = jnp.tanh(
            jnp.dot(ctxf, Wv_ref[:U, :], preferred_element_type=jnp.float32)
            + jnp.dot(ctxb, Wv_ref[U:2 * U, :], preferred_element_type=jnp.float32)
            + jnp.dot(htf, Wv_ref[2 * U:3 * U, :], preferred_element_type=jnp.float32)
            + jnp.dot(htb, Wv_ref[3 * U:, :], preferred_element_type=jnp.float32))
        logits = (jnp.dot(z1, Wd_ref[...], preferred_element_type=jnp.float32)
                  + bd_ref[...])
        mx = jnp.max(logits, axis=1, keepdims=True)
        ex = jnp.exp(logits - mx)
        out_ref[...] = ex / jnp.sum(ex, axis=1, keepdims=True)


def _attention(hf, hb, ht, Wv, Wd, bd):
    grid = (T // TW,)
    ones = jnp.ones((2 * U, U), jnp.bfloat16)
    return pl.pallas_call(
        _attn_body,
        grid=grid,
        in_specs=[
            pl.BlockSpec((TW, B, U), lambda t: (t, 0, 0)),
            pl.BlockSpec((TW, B, U), lambda t: (t, 0, 0)),
            pl.BlockSpec((B, 2 * U), lambda t: (0, 0)),
            pl.BlockSpec((2 * U, U), lambda t: (0, 0)),
            pl.BlockSpec((4 * U, U), lambda t: (0, 0)),
            pl.BlockSpec((U, 2), lambda t: (0, 0)),
            pl.BlockSpec((1, 2), lambda t: (0, 0)),
        ],
        out_specs=pl.BlockSpec((B, 2), lambda t: (0, 0)),
        out_shape=jax.ShapeDtypeStruct((B, 2), jnp.float32),
        scratch_shapes=[
            pltpu.VMEM((B, U), jnp.float32),
            pltpu.VMEM((B, U), jnp.float32),
            pltpu.VMEM((B, U), jnp.float32),
            pltpu.VMEM((B, 2 * U), jnp.bfloat16),
        ],
        compiler_params=pltpu.CompilerParams(
            dimension_semantics=("arbitrary",)),
    )(hf, hb, ht, ones, Wv, Wd, bd.reshape(1, -1))


def kernel(inputs, emb, Wf, Uf, bf, Wb, Ub, bb, Wv, Wd, bd):
    idx = jnp.swapaxes(inputs.astype(jnp.int32), 0, 1).reshape(-1)
    pad = EMB_P - EMB_D
    emb_p = jnp.pad(emb, ((0, 0), (0, pad)))
    gate_scale = jnp.concatenate(
        [jnp.full((2 * U,), 0.5), jnp.ones((U,)), jnp.full((U,), 0.5)]
    ).astype(jnp.float32)
    Wcf = (jnp.concatenate([jnp.pad(Wf, ((0, pad), (0, 0))), Uf], axis=0)
           * gate_scale).astype(jnp.bfloat16)
    Wcb = (jnp.concatenate([jnp.pad(Wb, ((0, pad), (0, 0))), Ub], axis=0)
           * gate_scale).astype(jnp.bfloat16)
    x = _sc_gather(emb_p, idx).reshape(T, B, EMB_P)
    hf, hb, ht = _bilstm(x, Wcf, bf * gate_scale, Wcb, bb * gate_scale)
    out = _attention(hf, hb, ht, Wv, Wd, bd)
    return out


# speedup vs baseline: 5.7228x; 1.0185x over previous
import functools

import jax
import jax.numpy as jnp
from jax import lax
from jax.experimental import pallas as pl
from jax.experimental.pallas import tpu as pltpu
from jax.experimental.pallas import tpu_sc as plsc

B = 1024
T = 200
EMB_D = 64
EMB_P = 128
U = 128

_NC = 2
_NS = 16
_NW = _NC * _NS


def _sc_gather(table, idx):
    n = idx.shape[0]
    d = table.shape[1]
    dt = table.dtype
    per_w = n // _NW
    chunk = 800 if dt == jnp.bfloat16 else 400
    n_chunks = per_w // chunk
    mesh = plsc.VectorSubcoreMesh(core_axis_name="c", subcore_axis_name="s")

    @functools.partial(
        pl.kernel,
        mesh=mesh,
        out_type=jax.ShapeDtypeStruct((n, d), dt),
        scratch_types=[
            pltpu.VMEM((per_w,), jnp.int32),
            pltpu.VMEM((chunk, d), dt),
            pltpu.VMEM((chunk, d), dt),
            pltpu.SemaphoreType.DMA,
            pltpu.SemaphoreType.DMA,
        ],
    )
    def k(table_hbm, idx_hbm, out_hbm, idx_v, rows_a, rows_b, sem_a, sem_b):
        wid = lax.axis_index("s") * _NC + lax.axis_index("c")
        base = wid * per_w
        pltpu.sync_copy(idx_hbm.at[pl.ds(base, per_w)], idx_v)
        bufs = (rows_a, rows_b)
        sems = (sem_a, sem_b)
        cps = [None, None]
        for c in range(n_chunks):
            s = c % 2
            cps[s] = pltpu.async_copy(
                table_hbm.at[idx_v.at[pl.ds(c * chunk, chunk)]], bufs[s], sems[s])
            if c > 0:
                cps[1 - s].wait()
                pltpu.sync_copy(bufs[1 - s],
                                out_hbm.at[pl.ds(base + (c - 1) * chunk, chunk)])
        cps[(n_chunks - 1) % 2].wait()
        pltpu.sync_copy(bufs[(n_chunks - 1) % 2],
                        out_hbm.at[pl.ds(base + (n_chunks - 1) * chunk, chunk)])

    return k(table, idx)


def _cell(x_ref, xc_s, c_bf, Wcat_ref, b_ref):
    xc_s[:, :EMB_P] = x_ref[0].astype(jnp.bfloat16)
    z = jnp.dot(xc_s[...], Wcat_ref[...],
                preferred_element_type=jnp.float32).astype(jnp.bfloat16)
    z = z + b_ref[...]
    half = jnp.bfloat16(0.5)
    i = half * jnp.tanh(z[:, :U]) + half
    f = half * jnp.tanh(z[:, U:2 * U]) + half
    g = jnp.tanh(z[:, 2 * U:3 * U])
    o = half * jnp.tanh(z[:, 3 * U:]) + half
    c2 = f * c_bf + i * g
    h2 = o * jnp.tanh(c2)
    xc_s[:, EMB_P:] = h2
    return h2, c2


def _scan_body(xf_ref, xb_ref, Wf_ref, bf_ref, Wb_ref, bb_ref,
               hf_out, hb_out, ht_out, xcf_s, cf_s, xcb_s, cb_s):
    t = pl.program_id(0)

    @pl.when(t == 0)
    def _init():
        xcf_s[:, EMB_P:] = jnp.zeros((B, U), jnp.bfloat16)
        cf_s[...] = jnp.zeros_like(cf_s)
        xcb_s[:, EMB_P:] = jnp.zeros((B, U), jnp.bfloat16)
        cb_s[...] = jnp.zeros_like(cb_s)

    h2f, c2f = _cell(xf_ref, xcf_s, cf_s[...], Wf_ref, bf_ref)
    h2b, c2b = _cell(xb_ref, xcb_s, cb_s[...], Wb_ref, bb_ref)
    cf_s[...] = c2f
    cb_s[...] = c2b
    hf_out[0] = h2f
    hb_out[0] = h2b

    @pl.when(t == 0)
    def _ht0():
        ht_out[:, :U] = h2f
        ht_out[:, U:] = h2b

    @pl.when(t > 0)
    def _htn():
        ht_out[:, :U] = jnp.maximum(ht_out[:, :U], h2f)
        ht_out[:, U:] = jnp.maximum(ht_out[:, U:], h2b)


def _bilstm(x, Wcf, bf, Wcb, bb):
    grid = (T,)
    return pl.pallas_call(
        _scan_body,
        grid=grid,
        in_specs=[
            pl.BlockSpec((1, B, EMB_P), lambda t: (t, 0, 0)),
            pl.BlockSpec((1, B, EMB_P), lambda t: (T - 1 - t, 0, 0)),
            pl.BlockSpec((EMB_P + U, 4 * U), lambda t: (0, 0)),
            pl.BlockSpec((1, 4 * U), lambda t: (0, 0)),
            pl.BlockSpec((EMB_P + U, 4 * U), lambda t: (0, 0)),
            pl.BlockSpec((1, 4 * U), lambda t: (0, 0)),
        ],
        out_specs=[
            pl.BlockSpec((1, B, U), lambda t: (t, 0, 0)),
            pl.BlockSpec((1, B, U), lambda t: (T - 1 - t, 0, 0)),
            pl.BlockSpec((B, 2 * U), lambda t: (0, 0)),
        ],
        out_shape=[
            jax.ShapeDtypeStruct((T, B, U), jnp.bfloat16),
            jax.ShapeDtypeStruct((T, B, U), jnp.bfloat16),
            jax.ShapeDtypeStruct((B, 2 * U), jnp.bfloat16),
        ],
        scratch_shapes=[
            pltpu.VMEM((B, EMB_P + U), jnp.bfloat16),
            pltpu.VMEM((B, U), jnp.bfloat16),
            pltpu.VMEM((B, EMB_P + U), jnp.bfloat16),
            pltpu.VMEM((B, U), jnp.bfloat16),
        ],
        compiler_params=pltpu.CompilerParams(
            dimension_semantics=("arbitrary",)),
    )(x, x, Wcf, bf.reshape(1, -1).astype(jnp.bfloat16),
      Wcb, bb.reshape(1, -1).astype(jnp.bfloat16))


TW = 8


def _attn_body(hf_ref, hb_ref, ht_ref, ones_ref, Wv_ref, Wd_ref, bd_ref,
               out_ref, S_s, Cf_s, Cb_s, q_s):
    tb = pl.program_id(0)

    @pl.when(tb == 0)
    def _zero():
        S_s[...] = jnp.zeros_like(S_s)
        Cf_s[...] = jnp.zeros_like(Cf_s)
        Cb_s[...] = jnp.zeros_like(Cb_s)

    es, hfs, hbs = [], [], []
    for k in range(TW):
        hfk = hf_ref[k]
        hbk = hb_ref[k]
        q_s[:, :U] = hfk * ht_ref[:, :U]
        q_s[:, U:] = hbk * ht_ref[:, U:]
        s = jnp.dot(q_s[...], ones_ref[...],
                    preferred_element_type=jnp.float32)
        es.append(jnp.exp(jnp.minimum(s, 80.0)))
        hfs.append(hfk.astype(jnp.float32))
        hbs.append(hbk.astype(jnp.float32))

    S_s[...] = S_s[...] + sum(es)
    Cf_s[...] = Cf_s[...] + sum(e * h for e, h in zip(es, hfs))
    Cb_s[...] = Cb_s[...] + sum(e * h for e, h in zip(es, hbs))

    @pl.when(tb == T // TW - 1)
    def _head():
        htf = ht_ref[:, :U].astype(jnp.float32)
        htb = ht_ref[:, U:].astype(jnp.float32)
        inv = 1.0 / S_s[...]
        ctxf = Cf_s[...] * inv
        ctxb = Cb_s[...] * inv
        z1 = jnp.tanh(
            jnp.dot(ctxf, Wv_ref[:U, :], preferred_element_type=jnp.float32)
            + jnp.dot(ctxb, Wv_ref[U:2 * U, :], preferred_element_type=jnp.float32)
            + jnp.dot(htf, Wv_ref[2 * U:3 * U, :], preferred_element_type=jnp.float32)
            + jnp.dot(htb, Wv_ref[3 * U:, :], preferred_element_type=jnp.float32))
        logits = (jnp.dot(z1, Wd_ref[...], preferred_element_type=jnp.float32)
                  + bd_ref[...])
        mx = jnp.max(logits, axis=1, keepdims=True)
        ex = jnp.exp(logits - mx)
        out_ref[...] = ex / jnp.sum(ex, axis=1, keepdims=True)


def _attention(hf, hb, ht, Wv, Wd, bd):
    grid = (T // TW,)
    ones = jnp.ones((2 * U, U), jnp.bfloat16)
    return pl.pallas_call(
        _attn_body,
        grid=grid,
        in_specs=[
            pl.BlockSpec((TW, B, U), lambda t: (t, 0, 0)),
            pl.BlockSpec((TW, B, U), lambda t: (t, 0, 0)),
            pl.BlockSpec((B, 2 * U), lambda t: (0, 0)),
            pl.BlockSpec((2 * U, U), lambda t: (0, 0)),
            pl.BlockSpec((4 * U, U), lambda t: (0, 0)),
            pl.BlockSpec((U, 2), lambda t: (0, 0)),
            pl.BlockSpec((1, 2), lambda t: (0, 0)),
        ],
        out_specs=pl.BlockSpec((B, 2), lambda t: (0, 0)),
        out_shape=jax.ShapeDtypeStruct((B, 2), jnp.float32),
        scratch_shapes=[
            pltpu.VMEM((B, U), jnp.float32),
            pltpu.VMEM((B, U), jnp.float32),
            pltpu.VMEM((B, U), jnp.float32),
            pltpu.VMEM((B, 2 * U), jnp.bfloat16),
        ],
        compiler_params=pltpu.CompilerParams(
            dimension_semantics=("arbitrary",)),
    )(hf, hb, ht, ones, Wv, Wd, bd.reshape(1, -1))


def kernel(inputs, emb, Wf, Uf, bf, Wb, Ub, bb, Wv, Wd, bd):
    idx = jnp.swapaxes(inputs.astype(jnp.int32), 0, 1).reshape(-1)
    pad = EMB_P - EMB_D
    emb_p = jnp.pad(emb, ((0, 0), (0, pad)))
    gate_scale = jnp.concatenate(
        [jnp.full((2 * U,), 0.5), jnp.ones((U,)), jnp.full((U,), 0.5)]
    ).astype(jnp.float32)
    Wcf = (jnp.concatenate([jnp.pad(Wf, ((0, pad), (0, 0))), Uf], axis=0)
           * gate_scale).astype(jnp.bfloat16)
    Wcb = (jnp.concatenate([jnp.pad(Wb, ((0, pad), (0, 0))), Ub], axis=0)
           * gate_scale).astype(jnp.bfloat16)
    x = _sc_gather(emb_p, idx).reshape(T, B, EMB_P)
    hf, hb, ht = _bilstm(x, Wcf, bf * gate_scale, Wcb, bb * gate_scale)
    out = _attention(hf, hb, ht, Wv, Wd, bd)
    return out
